# baseline (speedup 1.0000x reference)
import functools

import jax
import jax.numpy as jnp
from jax import lax
from jax.experimental import pallas as pl
from jax.experimental.pallas import tpu as pltpu
from jax.experimental.pallas import tpu_sc as plsc

HC = 64
NNODE = 100000
E = 1200000
NC, NS = 2, 16
BLK = 256
NSLOT = 6
EPAD = 1228800
NBT = EPAD // BLK
EDGES_PER_TILE = EPAD // NS
NB = EDGES_PER_TILE // BLK
ACC_ROWS = 100096
ROWS_PER_TILE = ACC_ROWS // NS
DUMMY = NNODE


def _sc_body(do_cnt, *refs):
    if do_cnt:
        (yf_u, yf_i, cq_u, cq_i, zeros_h, ones_h,
         sums_item, sums_user, cnt_item, cnt_user,
         acc, *rest) = refs
    else:
        (yf_u, yf_i, cq_u, cq_i, zeros_h, ones_h,
         sums_item, sums_user,
         acc, *rest) = refs
    cidx = tuple(rest[0:NSLOT])
    rows = tuple(rest[NSLOT:2 * NSLOT])
    ga = tuple(rest[2 * NSLOT:3 * NSLOT])
    sb = tuple(rest[3 * NSLOT:4 * NSLOT])

    c = lax.axis_index("c")
    s = lax.axis_index("s")
    my_rows = pl.ds(s * ROWS_PER_TILE, ROWS_PER_TILE)
    blk0 = s * NB

    def seg_pass(cq, yf, out3, q):
        pltpu.sync_copy(zeros_h, acc.at[my_rows])
        plsc.subcore_barrier()

        def istart(b, k):
            pltpu.async_copy(cq.at[q, blk0 + b], cidx[k], ga[k])

        def gstart(k):
            pltpu.async_copy(yf.at[cidx[k].at[0]], rows[k], ga[k])

        def sstart(k):
            pltpu.async_copy(rows[k], acc.at[cidx[k].at[1]], sb[k], add=True)

        def iwait(k):
            pltpu.make_async_copy(cq.at[q, 0], cidx[k], ga[k]).wait()

        def gwait(k):
            pltpu.make_async_copy(yf.at[cidx[k].at[0]], rows[k], ga[k]).wait()

        def swait(k):
            pltpu.make_async_copy(rows[k], acc.at[cidx[k].at[1]],
                                  sb[k]).wait()

        istart(0, 0)

        GLAG = 3

        def body(it, carry):
            for u in range(NSLOT):
                b = it * NSLOT + u
                k = u
                kp = (u + 1) % NSLOT
                kg = (u + NSLOT - GLAG) % NSLOT
                iwait(k)
                gstart(k)

                @pl.when(b >= GLAG)
                def _():
                    gwait(kg)
                    sstart(kg)

                @pl.when(b >= NSLOT - 1)
                def _():
                    swait(kp)

                @pl.when(b + 1 < NB)
                def _():
                    istart(b + 1, kp)
            return carry

        lax.fori_loop(0, NB // NSLOT, body, 0)
        for b in range(NB - GLAG, NB):
            gwait(b % NSLOT)
            sstart(b % NSLOT)
        for b in range(NB - NSLOT + 1, NB):
            swait(b % NSLOT)
        plsc.subcore_barrier()
        pltpu.sync_copy(acc.at[my_rows], out3.at[q, my_rows])

    for p in range(2):
        q = c * 2 + p
        seg_pass(cq_u, yf_u, sums_item, q)
        seg_pass(cq_i, yf_i, sums_user, q)

    if do_cnt:
        def cnt_pass(cq, out2):
            pltpu.sync_copy(ones_h, rows[0])
            pltpu.sync_copy(zeros_h, acc.at[my_rows])
            plsc.subcore_barrier()

            def sstart(k):
                pltpu.async_copy(rows[0], acc.at[cidx[k].at[1]],
                                 sb[k], add=True)

            def swait(k):
                pltpu.make_async_copy(rows[0], acc.at[cidx[k].at[1]],
                                      sb[k]).wait()

            def body(it, carry):
                for u in range(NSLOT):
                    b = it * NSLOT + u
                    k = u

                    @pl.when(b >= NSLOT)
                    def _():
                        swait(k)

                    pltpu.sync_copy(cq.at[0, blk0 + b], cidx[k])
                    sstart(k)
                return carry

            lax.fori_loop(0, NB // NSLOT, body, 0)
            for k in range(NSLOT):
                swait(k)
            plsc.subcore_barrier()
            pltpu.sync_copy(acc.at[my_rows], out2.at[my_rows])

        @pl.when(c == 0)
        def _():
            cnt_pass(cq_u, cnt_item)

        @pl.when(c == 1)
        def _():
            cnt_pass(cq_i, cnt_user)


def _make_sc(do_cnt):
    outs = [jax.ShapeDtypeStruct((4, ACC_ROWS, 16), jnp.float32)] * 2
    if do_cnt:
        outs += [jax.ShapeDtypeStruct((ACC_ROWS, 16), jnp.float32)] * 2
    mesh = plsc.VectorSubcoreMesh(
        core_axis_name="c", subcore_axis_name="s",
        num_cores=NC, num_subcores=NS)
    return pl.kernel(
        functools.partial(_sc_body, do_cnt),
        out_type=tuple(outs),
        mesh=mesh,
        scratch_types=[
            pltpu.VMEM_SHARED((ACC_ROWS, 16), jnp.float32),
            *[pltpu.VMEM((2, BLK), jnp.int32) for _ in range(NSLOT)],
            *[pltpu.VMEM((BLK, 16), jnp.float32) for _ in range(NSLOT)],
            *[pltpu.SemaphoreType.DMA for _ in range(2 * NSLOT)]
        ],
        compiler_params=pltpu.CompilerParams(use_tc_tiling_on_sc=False),
    )


_sc_l0 = _make_sc(True)
_sc_l1 = _make_sc(False)


def _mm_body(x_ref, w_ref, o_ref):
    o_ref[...] = jnp.dot(x_ref[...], w_ref[...],
                         preferred_element_type=jnp.float32)


def _mm(x, w):
    R = 2000
    return pl.pallas_call(
        _mm_body,
        grid=(NNODE // R,),
        in_specs=[pl.BlockSpec((R, HC), lambda i: (i, 0)),
                  pl.BlockSpec((HC, HC), lambda i: (0, 0))],
        out_specs=pl.BlockSpec((R, HC), lambda i: (i, 0)),
        out_shape=jax.ShapeDtypeStruct((NNODE, HC), jnp.float32),
    )(x, w)


def _post_body(relu, s_ref, c_ref, x_ref, wr_ref, b_ref, o_ref):
    sm = s_ref[...]
    m = jnp.concatenate([sm[0], sm[1], sm[2], sm[3]], axis=1)
    cnt = c_ref[...][:, 0:1]
    mean = m / jnp.maximum(cnt, 1.0)
    o = mean + b_ref[...] + jnp.dot(x_ref[...], wr_ref[...],
                                    preferred_element_type=jnp.float32)
    if relu:
        o = jnp.maximum(o, 0.0)
    o_ref[...] = o


def _post(sums, cnt, x, wr, b, relu):
    R = 2000
    return pl.pallas_call(
        functools.partial(_post_body, relu),
        grid=(NNODE // R,),
        in_specs=[pl.BlockSpec((4, R, 16), lambda i: (0, i, 0)),
                  pl.BlockSpec((R, 16), lambda i: (i, 0)),
                  pl.BlockSpec((R, HC), lambda i: (i, 0)),
                  pl.BlockSpec((HC, HC), lambda i: (0, 0)),
                  pl.BlockSpec((1, HC), lambda i: (0, 0))],
        out_specs=pl.BlockSpec((R, HC), lambda i: (i, 0)),
        out_shape=jax.ShapeDtypeStruct((NNODE, HC), jnp.float32),
    )(sums, cnt, x, wr, b)


def _prep(ei):
    src, dst = ei[0], ei[1]
    srcp = jnp.concatenate([src, jnp.zeros((EPAD - E,), jnp.int32)])
    dstp = jnp.concatenate([dst, jnp.full((EPAD - E,), DUMMY, jnp.int32)])
    srcq = (srcp * 4)[None, :] + jnp.arange(4, dtype=jnp.int32)[:, None]
    a = srcq.reshape(4, NBT, 1, BLK)
    b = jnp.broadcast_to(dstp.reshape(1, NBT, 1, BLK), (4, NBT, 1, BLK))
    return jnp.concatenate([a, b], axis=2)


def kernel(emb_user, emb_item, params, edge_index_user_rates_item,
           edge_index_item_rated_by_user):
    cq_u = _prep(edge_index_user_rates_item)
    cq_i = _prep(edge_index_item_rated_by_user)
    zeros_h = jnp.zeros((ROWS_PER_TILE, 16), jnp.float32)
    ones_h = jnp.ones((BLK, 16), jnp.float32)

    xu, xi = emb_user, emb_item
    cnt_item = cnt_user = None
    for l in range(2):
        pu = params["l%d_rates" % l]
        pi = params["l%d_rated_by" % l]
        y_u = _mm(xu, pu["W_l"]).reshape(4 * NNODE, 16)
        y_i = _mm(xi, pi["W_l"]).reshape(4 * NNODE, 16)
        if l == 0:
            sums_item, sums_user, cnt_item, cnt_user = _sc_l0(
                y_u, y_i, cq_u, cq_i, zeros_h, ones_h)
        else:
            sums_item, sums_user = _sc_l1(
                y_u, y_i, cq_u, cq_i, zeros_h, ones_h)
        new_xi = _post(sums_item, cnt_item, xi, pu["W_r"],
                       pu["b_l"].reshape(1, HC), relu=(l == 0))
        new_xu = _post(sums_user, cnt_user, xu, pi["W_r"],
                       pi["b_l"].reshape(1, HC), relu=(l == 0))
        xu, xi = new_xu, new_xi
    return (xu, xi)


# speedup vs baseline: 1.0122x; 1.0122x over previous
import functools

import jax
import jax.numpy as jnp
from jax import lax
from jax.experimental import pallas as pl
from jax.experimental.pallas import tpu as pltpu
from jax.experimental.pallas import tpu_sc as plsc

HC = 64
NNODE = 100000
E = 1200000
NC, NS = 2, 16
BLK = 384
NSLOT = 4
EPAD = 1228800
NBT = EPAD // BLK
EDGES_PER_TILE = EPAD // NS
NB = EDGES_PER_TILE // BLK
ACC_ROWS = 100096
ROWS_PER_TILE = ACC_ROWS // NS
DUMMY = NNODE


def _sc_body(do_cnt, *refs):
    if do_cnt:
        (yf_u, yf_i, cq_u, cq_i, zeros_h, ones_h,
         sums_item, sums_user, cnt_item, cnt_user,
         acc, *rest) = refs
    else:
        (yf_u, yf_i, cq_u, cq_i, zeros_h, ones_h,
         sums_item, sums_user,
         acc, *rest) = refs
    cidx = tuple(rest[0:NSLOT])
    rows = tuple(rest[NSLOT:2 * NSLOT])
    ga = tuple(rest[2 * NSLOT:3 * NSLOT])
    sb = tuple(rest[3 * NSLOT:4 * NSLOT])

    c = lax.axis_index("c")
    s = lax.axis_index("s")
    my_rows = pl.ds(s * ROWS_PER_TILE, ROWS_PER_TILE)
    blk0 = s * NB

    def seg_pass(cq, yf, out3, q):
        pltpu.sync_copy(zeros_h, acc.at[my_rows])
        plsc.subcore_barrier()

        def istart(b, k):
            pltpu.async_copy(cq.at[q, blk0 + b], cidx[k], ga[k])

        def gstart(k):
            pltpu.async_copy(yf.at[cidx[k].at[0]], rows[k], ga[k])

        def sstart(k):
            pltpu.async_copy(rows[k], acc.at[cidx[k].at[1]], sb[k], add=True)

        def iwait(k):
            pltpu.make_async_copy(cq.at[q, 0], cidx[k], ga[k]).wait()

        def gwait(k):
            pltpu.make_async_copy(yf.at[cidx[k].at[0]], rows[k], ga[k]).wait()

        def swait(k):
            pltpu.make_async_copy(rows[k], acc.at[cidx[k].at[1]],
                                  sb[k]).wait()

        istart(0, 0)

        GLAG = 2

        def body(it, carry):
            for u in range(NSLOT):
                b = it * NSLOT + u
                k = u
                kp = (u + 1) % NSLOT
                kg = (u + NSLOT - GLAG) % NSLOT
                iwait(k)
                gstart(k)

                @pl.when(b >= GLAG)
                def _():
                    gwait(kg)
                    sstart(kg)

                @pl.when(b >= NSLOT - 1)
                def _():
                    swait(kp)

                @pl.when(b + 1 < NB)
                def _():
                    istart(b + 1, kp)
            return carry

        lax.fori_loop(0, NB // NSLOT, body, 0)
        for b in range(NB - GLAG, NB):
            gwait(b % NSLOT)
            sstart(b % NSLOT)
        for b in range(NB - NSLOT + 1, NB):
            swait(b % NSLOT)
        plsc.subcore_barrier()
        pltpu.sync_copy(acc.at[my_rows], out3.at[q, my_rows])

    for p in range(2):
        q = c * 2 + p
        seg_pass(cq_u, yf_u, sums_item, q)
        seg_pass(cq_i, yf_i, sums_user, q)

    if do_cnt:
        def cnt_pass(cq, out2):
            pltpu.sync_copy(ones_h, rows[0])
            pltpu.sync_copy(zeros_h, acc.at[my_rows])
            plsc.subcore_barrier()

            def sstart(k):
                pltpu.async_copy(rows[0], acc.at[cidx[k].at[1]],
                                 sb[k], add=True)

            def swait(k):
                pltpu.make_async_copy(rows[0], acc.at[cidx[k].at[1]],
                                      sb[k]).wait()

            def body(it, carry):
                for u in range(NSLOT):
                    b = it * NSLOT + u
                    k = u

                    @pl.when(b >= NSLOT)
                    def _():
                        swait(k)

                    pltpu.sync_copy(cq.at[0, blk0 + b], cidx[k])
                    sstart(k)
                return carry

            lax.fori_loop(0, NB // NSLOT, body, 0)
            for k in range(NSLOT):
                swait(k)
            plsc.subcore_barrier()
            pltpu.sync_copy(acc.at[my_rows], out2.at[my_rows])

        @pl.when(c == 0)
        def _():
            cnt_pass(cq_u, cnt_item)

        @pl.when(c == 1)
        def _():
            cnt_pass(cq_i, cnt_user)


def _make_sc(do_cnt):
    outs = [jax.ShapeDtypeStruct((4, ACC_ROWS, 16), jnp.float32)] * 2
    if do_cnt:
        outs += [jax.ShapeDtypeStruct((ACC_ROWS, 16), jnp.float32)] * 2
    mesh = plsc.VectorSubcoreMesh(
        core_axis_name="c", subcore_axis_name="s",
        num_cores=NC, num_subcores=NS)
    return pl.kernel(
        functools.partial(_sc_body, do_cnt),
        out_type=tuple(outs),
        mesh=mesh,
        scratch_types=[
            pltpu.VMEM_SHARED((ACC_ROWS, 16), jnp.float32),
            *[pltpu.VMEM((2, BLK), jnp.int32) for _ in range(NSLOT)],
            *[pltpu.VMEM((BLK, 16), jnp.float32) for _ in range(NSLOT)],
            *[pltpu.SemaphoreType.DMA for _ in range(2 * NSLOT)]
        ],
        compiler_params=pltpu.CompilerParams(use_tc_tiling_on_sc=False),
    )


_sc_l0 = _make_sc(True)
_sc_l1 = _make_sc(False)


def _mm_body(x_ref, w_ref, o_ref):
    o_ref[...] = jnp.dot(x_ref[...], w_ref[...],
                         preferred_element_type=jnp.float32)


def _mm(x, w):
    R = 2000
    return pl.pallas_call(
        _mm_body,
        grid=(NNODE // R,),
        in_specs=[pl.BlockSpec((R, HC), lambda i: (i, 0)),
                  pl.BlockSpec((HC, HC), lambda i: (0, 0))],
        out_specs=pl.BlockSpec((R, HC), lambda i: (i, 0)),
        out_shape=jax.ShapeDtypeStruct((NNODE, HC), jnp.float32),
    )(x, w)


def _post_body(relu, s_ref, c_ref, x_ref, wr_ref, b_ref, o_ref):
    sm = s_ref[...]
    m = jnp.concatenate([sm[0], sm[1], sm[2], sm[3]], axis=1)
    cnt = c_ref[...][:, 0:1]
    mean = m / jnp.maximum(cnt, 1.0)
    o = mean + b_ref[...] + jnp.dot(x_ref[...], wr_ref[...],
                                    preferred_element_type=jnp.float32)
    if relu:
        o = jnp.maximum(o, 0.0)
    o_ref[...] = o


def _post(sums, cnt, x, wr, b, relu):
    R = 2000
    return pl.pallas_call(
        functools.partial(_post_body, relu),
        grid=(NNODE // R,),
        in_specs=[pl.BlockSpec((4, R, 16), lambda i: (0, i, 0)),
                  pl.BlockSpec((R, 16), lambda i: (i, 0)),
                  pl.BlockSpec((R, HC), lambda i: (i, 0)),
                  pl.BlockSpec((HC, HC), lambda i: (0, 0)),
                  pl.BlockSpec((1, HC), lambda i: (0, 0))],
        out_specs=pl.BlockSpec((R, HC), lambda i: (i, 0)),
        out_shape=jax.ShapeDtypeStruct((NNODE, HC), jnp.float32),
    )(sums, cnt, x, wr, b)


def _prep(ei):
    src, dst = ei[0], ei[1]
    srcp = jnp.concatenate([src, jnp.zeros((EPAD - E,), jnp.int32)])
    dstp = jnp.concatenate([dst, jnp.full((EPAD - E,), DUMMY, jnp.int32)])
    srcq = (srcp * 4)[None, :] + jnp.arange(4, dtype=jnp.int32)[:, None]
    a = srcq.reshape(4, NBT, 1, BLK)
    b = jnp.broadcast_to(dstp.reshape(1, NBT, 1, BLK), (4, NBT, 1, BLK))
    return jnp.concatenate([a, b], axis=2)


def kernel(emb_user, emb_item, params, edge_index_user_rates_item,
           edge_index_item_rated_by_user):
    cq_u = _prep(edge_index_user_rates_item)
    cq_i = _prep(edge_index_item_rated_by_user)
    zeros_h = jnp.zeros((ROWS_PER_TILE, 16), jnp.float32)
    ones_h = jnp.ones((BLK, 16), jnp.float32)

    xu, xi = emb_user, emb_item
    cnt_item = cnt_user = None
    for l in range(2):
        pu = params["l%d_rates" % l]
        pi = params["l%d_rated_by" % l]
        y_u = _mm(xu, pu["W_l"]).reshape(4 * NNODE, 16)
        y_i = _mm(xi, pi["W_l"]).reshape(4 * NNODE, 16)
        if l == 0:
            sums_item, sums_user, cnt_item, cnt_user = _sc_l0(
                y_u, y_i, cq_u, cq_i, zeros_h, ones_h)
        else:
            sums_item, sums_user = _sc_l1(
                y_u, y_i, cq_u, cq_i, zeros_h, ones_h)
        new_xi = _post(sums_item, cnt_item, xi, pu["W_r"],
                       pu["b_l"].reshape(1, HC), relu=(l == 0))
        new_xu = _post(sums_user, cnt_user, xu, pi["W_r"],
                       pi["b_l"].reshape(1, HC), relu=(l == 0))
        xu, xi = new_xu, new_xi
    return (xu, xi)
